# baseline (speedup 1.0000x reference)
import functools

import jax
import jax.numpy as jnp
from jax import lax
from jax.experimental import pallas as pl
from jax.experimental.pallas import tpu as pltpu
from jax.experimental.pallas import tpu_sc as plsc

NNODES = 10000
NEDGES = 320000
LATENT = 128
NODES_PAD = 10240

NC, NS, LANES = 2, 16, 16
NW = NC * NS
EPW = NEDGES // NW
CH = 80
NCH = EPW // CH
ROWS_T = NODES_PAD // NS
CW = 128

E_BLK = 1280
N_BLK = 2000

_F32 = jnp.float32


def _swish(x):
    return x * jax.nn.sigmoid(x)


def _layernorm(y, scale, bias):
    mu = jnp.mean(y, axis=-1, keepdims=True)
    d = y - mu
    var = jnp.mean(d * d, axis=-1, keepdims=True)
    return d * lax.rsqrt(var + 1e-5) * scale + bias


def _dot(a, b):
    return jnp.dot(a.astype(jnp.bfloat16), b.astype(jnp.bfloat16),
                   preferred_element_type=_F32)


def _mlp2_ln_body(x, w1, b1, w2, b2, lns, lnb, o):
    h = _swish(_dot(x[...], w1[...]) + b1[...])
    o[...] = _layernorm(_dot(h, w2[...]) + b2[...], lns[...], lnb[...])


def _mlp2_body(x, w1, b1, w2, b2, o):
    h = _swish(_dot(x[...], w1[...]) + b1[...])
    o[...] = _dot(h, w2[...]) + b2[...]


def _full(shape):
    return pl.BlockSpec(shape, lambda i: tuple(0 for _ in shape))


def _mlp2(x, p, blk):
    n, din = x.shape
    w1, w2 = p['Ws']
    b1, b2 = (b.reshape(1, -1) for b in p['bs'])
    args = [x, w1, b1, w2, b2]
    specs = [pl.BlockSpec((blk, din), lambda i: (i, 0)),
             _full(w1.shape), _full(b1.shape), _full(w2.shape), _full(b2.shape)]
    body = _mlp2_body
    if 'ln' in p:
        body = _mlp2_ln_body
        args += [p['ln']['scale'].reshape(1, -1), p['ln']['bias'].reshape(1, -1)]
        specs += [_full((1, LATENT)), _full((1, LATENT))]
    return pl.pallas_call(
        body,
        grid=(n // blk,),
        in_specs=specs,
        out_specs=pl.BlockSpec((blk, LATENT), lambda i: (i, 0)),
        out_shape=jax.ShapeDtypeStruct((n, LATENT), _F32),
    )(*args)


def _node_proj_body(nf, w1a, w1b, b1, oa, ob):
    x = nf[...]
    oa[...] = _dot(x, w1a[...]) + b1[...]
    ob[...] = _dot(x, w1b[...])


def _node_proj(nf, w1a, w1b, b1):
    return pl.pallas_call(
        _node_proj_body,
        grid=(NNODES // N_BLK,),
        in_specs=[pl.BlockSpec((N_BLK, LATENT), lambda i: (i, 0)),
                  _full((LATENT, LATENT)), _full((LATENT, LATENT)),
                  _full((1, LATENT))],
        out_specs=[pl.BlockSpec((N_BLK, LATENT), lambda i: (i, 0))] * 2,
        out_shape=[jax.ShapeDtypeStruct((NNODES, LATENT), _F32)] * 2,
    )(nf, w1a, w1b, b1.reshape(1, -1))


def _edge_step_body(g, ef, w1c, w2, b2, lns, lnb, o):
    h = _swish(g[...] + _dot(ef[...], w1c[...]))
    o[...] = _layernorm(_dot(h, w2[...]) + b2[...], lns[...], lnb[...])


def _edge_step(g, ef, w1c, p):
    n = g.shape[0]
    row = pl.BlockSpec((E_BLK, LATENT), lambda i: (i, 0))
    return pl.pallas_call(
        _edge_step_body,
        grid=(n // E_BLK,),
        in_specs=[row, row, _full((LATENT, LATENT)), _full((LATENT, LATENT)),
                  _full((1, LATENT)), _full((1, LATENT)), _full((1, LATENT))],
        out_specs=row,
        out_shape=jax.ShapeDtypeStruct((n, LATENT), _F32),
    )(g, ef, w1c, p['Ws'][1], p['bs'][1].reshape(1, -1),
      p['ln']['scale'].reshape(1, -1), p['ln']['bias'].reshape(1, -1))


def _node_mlp_body(nf, p00, p01, p10, p11, c0, c1,
                   wna, wnb, b1, w2, b2, lns, lnb, o):
    cnt = jnp.maximum(c0[...][:, :1] + c1[...][:, :1], 1.0)
    me = (p00[...] + p01[...] + p10[...] + p11[...]) / cnt
    h = _swish(_dot(nf[...], wna[...]) + _dot(me, wnb[...]) + b1[...])
    o[...] = _layernorm(_dot(h, w2[...]) + b2[...], lns[...], lnb[...])


def _node_mlp(nf, parts_a, parts_b, c0, c1, p):
    wn = p['Ws'][0]
    row = pl.BlockSpec((N_BLK, LATENT), lambda i: (i, 0))
    crow = pl.BlockSpec((N_BLK, CW), lambda i: (i, 0))
    return pl.pallas_call(
        _node_mlp_body,
        grid=(NNODES // N_BLK,),
        in_specs=[row, row, row, row, row, crow, crow,
                  _full((LATENT, LATENT)), _full((LATENT, LATENT)),
                  _full((1, LATENT)), _full((LATENT, LATENT)),
                  _full((1, LATENT)), _full((1, LATENT)), _full((1, LATENT))],
        out_specs=row,
        out_shape=jax.ShapeDtypeStruct((NNODES, LATENT), _F32),
    )(nf, parts_a[0, :NNODES], parts_a[1, :NNODES],
      parts_b[0, :NNODES], parts_b[1, :NNODES], c0, c1,
      wn[:LATENT], wn[LATENT:], p['bs'][0].reshape(1, -1),
      p['Ws'][1], p['bs'][1].reshape(1, -1),
      p['ln']['scale'].reshape(1, -1), p['ln']['bias'].reshape(1, -1))


_SC_MESH = plsc.VectorSubcoreMesh(core_axis_name="c", subcore_axis_name="s")


def _make_gather(nedges, ch):
    epw = nedges // NW
    nch = epw // ch

    @functools.partial(
        pl.kernel, mesh=_SC_MESH,
        out_type=jax.ShapeDtypeStruct((nedges, LATENT), _F32),
        scratch_types=[
            pltpu.VMEM((2, ch), jnp.int32),
            pltpu.VMEM((2, ch), jnp.int32),
            pltpu.VMEM((2, ch, LATENT), _F32),
            pltpu.VMEM((2, ch, LATENT), _F32),
            pltpu.SemaphoreType.DMA,
            pltpu.SemaphoreType.DMA,
        ],
    )
    def gather(nfa, nfb, snd, rcv, out, idxa2, idxb2, ra2, rb2, g0, g1):
        wid = lax.axis_index("s") * NC + lax.axis_index("c")
        sems = (g0, g1)

        def issue(j, b):
            base = wid * epw + j * ch
            pltpu.sync_copy(snd.at[pl.ds(base, ch)], idxa2.at[b])
            pltpu.sync_copy(rcv.at[pl.ds(base, ch)], idxb2.at[b])
            pltpu.async_copy(nfa.at[idxa2.at[b]], ra2.at[b], sems[b])
            pltpu.async_copy(nfb.at[idxb2.at[b]], rb2.at[b], sems[b])

        def process(j, b):
            pltpu.make_async_copy(nfa.at[idxa2.at[b]], ra2.at[b],
                                  sems[b]).wait()
            pltpu.make_async_copy(nfb.at[idxb2.at[b]], rb2.at[b],
                                  sems[b]).wait()

            def row(i, c):
                for q in range(LATENT // LANES):
                    sl = pl.ds(q * LANES, LANES)
                    ra2[b, i, sl] = ra2[b, i, sl] + rb2[b, i, sl]
                return c

            lax.fori_loop(0, ch, row, 0)
            pltpu.sync_copy(ra2.at[b], out.at[pl.ds(wid * epw + j * ch, ch)])

        issue(0, 0)

        def pair(k2, c):
            j0 = 2 * k2
            issue(j0 + 1, 1)
            process(j0, 0)
            issue(j0 + 2, 0)
            process(j0 + 1, 1)
            return c

        if nch % 2 == 1:
            lax.fori_loop(0, (nch - 1) // 2, pair, 0)
            process(nch - 1, 0)
        else:
            lax.fori_loop(0, nch // 2 - 1, pair, 0)
            issue(nch - 1, 1)
            process(nch - 2, 0)
            process(nch - 1, 1)

    return gather


def _make_scatter(nedges, ch):
    epw = nedges // NW
    nch = epw // ch

    @functools.partial(
        pl.kernel, mesh=_SC_MESH,
        out_type=jax.ShapeDtypeStruct((NC, NODES_PAD, LATENT), _F32),
        scratch_types=[
            pltpu.VMEM_SHARED((NODES_PAD, LATENT), _F32),
            pltpu.VMEM((2, ch), jnp.int32),
            pltpu.VMEM((2, ch, LATENT), _F32),
            pltpu.SemaphoreType.DMA,
            pltpu.SemaphoreType.DMA,
        ],
    )
    def scatter(rcv, vals, out, accum, idx2, rows2, v0, v1):
        cid = lax.axis_index("c")
        sid = lax.axis_index("s")
        wid = sid * NC + cid
        sems = (v0, v1)

        def zrow(i, c):
            for j in range(LATENT // LANES):
                rows2[0, i, pl.ds(j * LANES, LANES)] = jnp.zeros((LANES,),
                                                                 _F32)
            return c

        lax.fori_loop(0, ch, zrow, 0)
        for q in range(ROWS_T // ch):
            pltpu.sync_copy(rows2.at[0],
                            accum.at[pl.ds(sid * ROWS_T + q * ch, ch)])
        plsc.subcore_barrier()

        def issue(j, b):
            base = wid * epw + j * ch
            pltpu.sync_copy(rcv.at[pl.ds(base, ch)], idx2.at[b])
            pltpu.async_copy(vals.at[pl.ds(base, ch)], rows2.at[b], sems[b])

        def process(j, b):
            base = wid * epw + j * ch
            pltpu.make_async_copy(vals.at[pl.ds(base, ch)], rows2.at[b],
                                  sems[b]).wait()
            pltpu.sync_copy(rows2.at[b], accum.at[idx2.at[b]], add=True)

        issue(0, 0)

        def pair(k2, c):
            j0 = 2 * k2
            issue(j0 + 1, 1)
            process(j0, 0)
            issue(j0 + 2, 0)
            process(j0 + 1, 1)
            return c

        if nch % 2 == 1:
            lax.fori_loop(0, (nch - 1) // 2, pair, 0)
            process(nch - 1, 0)
        else:
            lax.fori_loop(0, nch // 2 - 1, pair, 0)
            issue(nch - 1, 1)
            process(nch - 2, 0)
            process(nch - 1, 1)
        plsc.subcore_barrier()
        for q in range(ROWS_T // ch):
            r0 = sid * ROWS_T + q * ch
            pltpu.sync_copy(accum.at[pl.ds(r0, ch)], rows2.at[0])
            pltpu.sync_copy(rows2.at[0], out.at[cid, pl.ds(r0, ch)])

    return scatter


NE_A = 192000
NE_B = NEDGES - NE_A
_sc_gather_a = _make_gather(NE_A, CH)
_sc_gather_b = _make_gather(NE_B, CH)
_sc_scatter_a = _make_scatter(NE_A, CH)
_sc_scatter_b = _make_scatter(NE_B, CH)


@functools.partial(
    pl.kernel, mesh=_SC_MESH,
    out_type=jax.ShapeDtypeStruct((NC, NODES_PAD, CW), _F32),
    scratch_types=[
        pltpu.VMEM_SHARED((NODES_PAD, CW), _F32),
        pltpu.VMEM((2, CH), jnp.int32),
        pltpu.VMEM((CH, CW), _F32),
        pltpu.SemaphoreType.DMA,
        pltpu.SemaphoreType.DMA,
    ],
)
def _sc_counts(rcv, out, accum, idx2, tmp, i0, i1):
    cid = lax.axis_index("c")
    sid = lax.axis_index("s")
    wid = sid * NC + cid
    sems = (i0, i1)

    def fill(val):
        def frow(i, c):
            for j in range(CW // LANES):
                tmp[i, pl.ds(j * LANES, LANES)] = jnp.full((LANES,), val, _F32)
            return c
        lax.fori_loop(0, CH, frow, 0)

    fill(0.0)
    for q in range(ROWS_T // CH):
        pltpu.sync_copy(tmp, accum.at[pl.ds(sid * ROWS_T + q * CH, CH)])
    plsc.subcore_barrier()
    fill(1.0)

    def issue(j, b):
        base = wid * EPW + j * CH
        pltpu.async_copy(rcv.at[pl.ds(base, CH)], idx2.at[b], sems[b])

    def process(j, b):
        base = wid * EPW + j * CH
        pltpu.make_async_copy(rcv.at[pl.ds(base, CH)], idx2.at[b],
                              sems[b]).wait()
        pltpu.sync_copy(tmp, accum.at[idx2.at[b]], add=True)

    issue(0, 0)

    def pair(k2, c):
        j0 = 2 * k2
        issue(j0 + 1, 1)
        process(j0, 0)
        issue(j0 + 2, 0)
        process(j0 + 1, 1)
        return c

    lax.fori_loop(0, (NCH - 1) // 2, pair, 0)
    process(NCH - 1, 0)
    plsc.subcore_barrier()
    for q in range(ROWS_T // CH):
        r0 = sid * ROWS_T + q * CH
        pltpu.sync_copy(accum.at[pl.ds(r0, CH)], tmp)
        pltpu.sync_copy(tmp, out.at[cid, pl.ds(r0, CH)])


def kernel(edge_idx, edge_features, node_features, params):
    snd = edge_idx[:, 0].astype(jnp.int32)
    rcv = edge_idx[:, 1].astype(jnp.int32)
    snd_h = (snd[:NE_A], snd[NE_A:])
    rcv_h = (rcv[:NE_A], rcv[NE_A:])
    gathers = (_sc_gather_a, _sc_gather_b)
    scatters = (_sc_scatter_a, _sc_scatter_b)

    ef_h = [_mlp2(edge_features[:NE_A], params['embed_edge'], E_BLK),
            _mlp2(edge_features[NE_A:], params['embed_edge'], E_BLK)]
    nf = _mlp2(node_features, params['embed_node'], N_BLK)

    counts = _sc_counts(rcv)
    c0 = counts[0, :NNODES]
    c1 = counts[1, :NNODES]

    for step in params['process']:
        w1 = step['edge']['Ws'][0]
        nfa, nfb = _node_proj(nf, w1[:LATENT], w1[LATENT:2 * LATENT],
                              step['edge']['bs'][0])
        g_h = [gathers[i](nfa, nfb, snd_h[i], rcv_h[i]) for i in range(2)]
        new_ef_h = [_edge_step(g_h[i], ef_h[i], w1[2 * LATENT:],
                               step['edge'])
                    for i in range(2)]
        parts_h = [scatters[i](rcv_h[i], new_ef_h[i]) for i in range(2)]
        nf = _node_mlp(nf, parts_h[0], parts_h[1], c0, c1, step['node'])
        ef_h = new_ef_h

    ef = jnp.concatenate([_mlp2(ef_h[0], params['edge_out'], E_BLK),
                          _mlp2(ef_h[1], params['edge_out'], E_BLK)], axis=0)
    nf = _mlp2(nf, params['node_out'], N_BLK)
    return ef, nf


# speedup vs baseline: 1.0511x; 1.0511x over previous
import functools

import jax
import jax.numpy as jnp
from jax import lax
from jax.experimental import pallas as pl
from jax.experimental.pallas import tpu as pltpu
from jax.experimental.pallas import tpu_sc as plsc

NNODES = 10000
NEDGES = 320000
LATENT = 128
NODES_PAD = 10240

NC, NS, LANES = 2, 16, 16
NW = NC * NS
EPW = NEDGES // NW
CH = 80
NCH = EPW // CH
ROWS_T = NODES_PAD // NS
CW = 128

E_BLK = 1280
N_BLK = 2000

_F32 = jnp.float32


def _swish(x):
    return x * jax.nn.sigmoid(x)


def _layernorm(y, scale, bias):
    mu = jnp.mean(y, axis=-1, keepdims=True)
    d = y - mu
    var = jnp.mean(d * d, axis=-1, keepdims=True)
    return d * lax.rsqrt(var + 1e-5) * scale + bias


def _dot(a, b):
    return jnp.dot(a.astype(jnp.bfloat16), b.astype(jnp.bfloat16),
                   preferred_element_type=_F32)


def _mlp2_ln_body(x, w1, b1, w2, b2, lns, lnb, o):
    h = _swish(_dot(x[...], w1[...]) + b1[...])
    o[...] = _layernorm(_dot(h, w2[...]) + b2[...], lns[...], lnb[...])


def _mlp2_body(x, w1, b1, w2, b2, o):
    h = _swish(_dot(x[...], w1[...]) + b1[...])
    o[...] = _dot(h, w2[...]) + b2[...]


def _full(shape):
    return pl.BlockSpec(shape, lambda i: tuple(0 for _ in shape))


def _mlp2(x, p, blk):
    n, din = x.shape
    w1, w2 = p['Ws']
    b1, b2 = (b.reshape(1, -1) for b in p['bs'])
    args = [x, w1, b1, w2, b2]
    specs = [pl.BlockSpec((blk, din), lambda i: (i, 0)),
             _full(w1.shape), _full(b1.shape), _full(w2.shape), _full(b2.shape)]
    body = _mlp2_body
    if 'ln' in p:
        body = _mlp2_ln_body
        args += [p['ln']['scale'].reshape(1, -1), p['ln']['bias'].reshape(1, -1)]
        specs += [_full((1, LATENT)), _full((1, LATENT))]
    return pl.pallas_call(
        body,
        grid=(n // blk,),
        in_specs=specs,
        out_specs=pl.BlockSpec((blk, LATENT), lambda i: (i, 0)),
        out_shape=jax.ShapeDtypeStruct((n, LATENT), _F32),
    )(*args)


def _node_proj_body(nf, w1a, w1b, b1, oa, ob):
    x = nf[...]
    oa[...] = _dot(x, w1a[...]) + b1[...]
    ob[...] = _dot(x, w1b[...])


def _node_proj(nf, w1a, w1b, b1):
    return pl.pallas_call(
        _node_proj_body,
        grid=(NNODES // N_BLK,),
        in_specs=[pl.BlockSpec((N_BLK, LATENT), lambda i: (i, 0)),
                  _full((LATENT, LATENT)), _full((LATENT, LATENT)),
                  _full((1, LATENT))],
        out_specs=[pl.BlockSpec((N_BLK, LATENT), lambda i: (i, 0))] * 2,
        out_shape=[jax.ShapeDtypeStruct((NNODES, LATENT), _F32)] * 2,
    )(nf, w1a, w1b, b1.reshape(1, -1))


def _edge_step_body(g, ef, w1c, w2, b2, lns, lnb, o):
    h = _swish(g[...] + _dot(ef[...], w1c[...]))
    o[...] = _layernorm(_dot(h, w2[...]) + b2[...], lns[...], lnb[...])


def _edge_step(g, ef, w1c, p):
    n = g.shape[0]
    row = pl.BlockSpec((E_BLK, LATENT), lambda i: (i, 0))
    return pl.pallas_call(
        _edge_step_body,
        grid=(n // E_BLK,),
        in_specs=[row, row, _full((LATENT, LATENT)), _full((LATENT, LATENT)),
                  _full((1, LATENT)), _full((1, LATENT)), _full((1, LATENT))],
        out_specs=row,
        out_shape=jax.ShapeDtypeStruct((n, LATENT), _F32),
    )(g, ef, w1c, p['Ws'][1], p['bs'][1].reshape(1, -1),
      p['ln']['scale'].reshape(1, -1), p['ln']['bias'].reshape(1, -1))


def _node_mlp(nf, parts_list, c0, c1, p):
    np_ = 2 * len(parts_list)

    def body(*refs):
        nf_r = refs[0]
        ps = refs[1:1 + np_]
        c0_r, c1_r = refs[1 + np_:3 + np_]
        wna, wnb, b1, w2, b2, lns, lnb = refs[3 + np_:-1]
        o = refs[-1]
        cnt = jnp.maximum(c0_r[...][:, :1] + c1_r[...][:, :1], 1.0)
        acc = ps[0][...]
        for pr in ps[1:]:
            acc = acc + pr[...]
        me = acc / cnt
        h = _swish(_dot(nf_r[...], wna[...]) + _dot(me, wnb[...]) + b1[...])
        o[...] = _layernorm(_dot(h, w2[...]) + b2[...], lns[...], lnb[...])

    wn = p['Ws'][0]
    row = pl.BlockSpec((N_BLK, LATENT), lambda i: (i, 0))
    crow = pl.BlockSpec((N_BLK, CW), lambda i: (i, 0))
    return pl.pallas_call(
        body,
        grid=(NNODES // N_BLK,),
        in_specs=[row] * (1 + np_) + [crow, crow,
                  _full((LATENT, LATENT)), _full((LATENT, LATENT)),
                  _full((1, LATENT)), _full((LATENT, LATENT)),
                  _full((1, LATENT)), _full((1, LATENT)), _full((1, LATENT))],
        out_specs=row,
        out_shape=jax.ShapeDtypeStruct((NNODES, LATENT), _F32),
    )(nf, *[pt[i, :NNODES] for pt in parts_list for i in range(2)],
      c0, c1,
      wn[:LATENT], wn[LATENT:], p['bs'][0].reshape(1, -1),
      p['Ws'][1], p['bs'][1].reshape(1, -1),
      p['ln']['scale'].reshape(1, -1), p['ln']['bias'].reshape(1, -1))


_SC_MESH = plsc.VectorSubcoreMesh(core_axis_name="c", subcore_axis_name="s")


def _make_gather(nedges, ch):
    epw = nedges // NW
    nch = epw // ch

    @functools.partial(
        pl.kernel, mesh=_SC_MESH,
        out_type=jax.ShapeDtypeStruct((nedges, LATENT), _F32),
        scratch_types=[
            pltpu.VMEM((2, ch), jnp.int32),
            pltpu.VMEM((2, ch), jnp.int32),
            pltpu.VMEM((2, ch, LATENT), _F32),
            pltpu.VMEM((2, ch, LATENT), _F32),
            pltpu.SemaphoreType.DMA,
            pltpu.SemaphoreType.DMA,
        ],
    )
    def gather(nfa, nfb, snd, rcv, out, idxa2, idxb2, ra2, rb2, g0, g1):
        wid = lax.axis_index("s") * NC + lax.axis_index("c")
        sems = (g0, g1)

        def issue(j, b):
            base = wid * epw + j * ch
            pltpu.sync_copy(snd.at[pl.ds(base, ch)], idxa2.at[b])
            pltpu.sync_copy(rcv.at[pl.ds(base, ch)], idxb2.at[b])
            pltpu.async_copy(nfa.at[idxa2.at[b]], ra2.at[b], sems[b])
            pltpu.async_copy(nfb.at[idxb2.at[b]], rb2.at[b], sems[b])

        def process(j, b):
            pltpu.make_async_copy(nfa.at[idxa2.at[b]], ra2.at[b],
                                  sems[b]).wait()
            pltpu.make_async_copy(nfb.at[idxb2.at[b]], rb2.at[b],
                                  sems[b]).wait()

            def row(i, c):
                for q in range(LATENT // LANES):
                    sl = pl.ds(q * LANES, LANES)
                    ra2[b, i, sl] = ra2[b, i, sl] + rb2[b, i, sl]
                return c

            lax.fori_loop(0, ch, row, 0)
            pltpu.sync_copy(ra2.at[b], out.at[pl.ds(wid * epw + j * ch, ch)])

        issue(0, 0)

        def pair(k2, c):
            j0 = 2 * k2
            issue(j0 + 1, 1)
            process(j0, 0)
            issue(j0 + 2, 0)
            process(j0 + 1, 1)
            return c

        if nch % 2 == 1:
            lax.fori_loop(0, (nch - 1) // 2, pair, 0)
            process(nch - 1, 0)
        else:
            lax.fori_loop(0, nch // 2 - 1, pair, 0)
            issue(nch - 1, 1)
            process(nch - 2, 0)
            process(nch - 1, 1)

    return gather


def _make_scatter(nedges, ch):
    epw = nedges // NW
    nch = epw // ch

    @functools.partial(
        pl.kernel, mesh=_SC_MESH,
        out_type=jax.ShapeDtypeStruct((NC, NODES_PAD, LATENT), _F32),
        scratch_types=[
            pltpu.VMEM_SHARED((NODES_PAD, LATENT), _F32),
            pltpu.VMEM((2, ch), jnp.int32),
            pltpu.VMEM((2, ch, LATENT), _F32),
            pltpu.SemaphoreType.DMA,
            pltpu.SemaphoreType.DMA,
        ],
    )
    def scatter(rcv, vals, out, accum, idx2, rows2, v0, v1):
        cid = lax.axis_index("c")
        sid = lax.axis_index("s")
        wid = sid * NC + cid
        sems = (v0, v1)

        def zrow(i, c):
            for j in range(LATENT // LANES):
                rows2[0, i, pl.ds(j * LANES, LANES)] = jnp.zeros((LANES,),
                                                                 _F32)
            return c

        lax.fori_loop(0, ch, zrow, 0)
        for q in range(ROWS_T // ch):
            pltpu.sync_copy(rows2.at[0],
                            accum.at[pl.ds(sid * ROWS_T + q * ch, ch)])
        plsc.subcore_barrier()

        def issue(j, b):
            base = wid * epw + j * ch
            pltpu.sync_copy(rcv.at[pl.ds(base, ch)], idx2.at[b])
            pltpu.async_copy(vals.at[pl.ds(base, ch)], rows2.at[b], sems[b])

        def process(j, b):
            base = wid * epw + j * ch
            pltpu.make_async_copy(vals.at[pl.ds(base, ch)], rows2.at[b],
                                  sems[b]).wait()
            pltpu.sync_copy(rows2.at[b], accum.at[idx2.at[b]], add=True)

        issue(0, 0)

        def pair(k2, c):
            j0 = 2 * k2
            issue(j0 + 1, 1)
            process(j0, 0)
            issue(j0 + 2, 0)
            process(j0 + 1, 1)
            return c

        if nch % 2 == 1:
            lax.fori_loop(0, (nch - 1) // 2, pair, 0)
            process(nch - 1, 0)
        else:
            lax.fori_loop(0, nch // 2 - 1, pair, 0)
            issue(nch - 1, 1)
            process(nch - 2, 0)
            process(nch - 1, 1)
        plsc.subcore_barrier()
        for q in range(ROWS_T // ch):
            r0 = sid * ROWS_T + q * ch
            pltpu.sync_copy(accum.at[pl.ds(r0, ch)], rows2.at[0])
            pltpu.sync_copy(rows2.at[0], out.at[cid, pl.ds(r0, ch)])

    return scatter


PARTS = (107520, 104960, 107520)
_PART_OFF = tuple(sum(PARTS[:i]) for i in range(len(PARTS) + 1))
_sc_gathers = tuple(_make_gather(n, CH) for n in PARTS)
_sc_scatters = tuple(_make_scatter(n, CH) for n in PARTS)


@functools.partial(
    pl.kernel, mesh=_SC_MESH,
    out_type=jax.ShapeDtypeStruct((NC, NODES_PAD, CW), _F32),
    scratch_types=[
        pltpu.VMEM_SHARED((NODES_PAD, CW), _F32),
        pltpu.VMEM((2, CH), jnp.int32),
        pltpu.VMEM((CH, CW), _F32),
        pltpu.SemaphoreType.DMA,
        pltpu.SemaphoreType.DMA,
    ],
)
def _sc_counts(rcv, out, accum, idx2, tmp, i0, i1):
    cid = lax.axis_index("c")
    sid = lax.axis_index("s")
    wid = sid * NC + cid
    sems = (i0, i1)

    def fill(val):
        def frow(i, c):
            for j in range(CW // LANES):
                tmp[i, pl.ds(j * LANES, LANES)] = jnp.full((LANES,), val, _F32)
            return c
        lax.fori_loop(0, CH, frow, 0)

    fill(0.0)
    for q in range(ROWS_T // CH):
        pltpu.sync_copy(tmp, accum.at[pl.ds(sid * ROWS_T + q * CH, CH)])
    plsc.subcore_barrier()
    fill(1.0)

    def issue(j, b):
        base = wid * EPW + j * CH
        pltpu.async_copy(rcv.at[pl.ds(base, CH)], idx2.at[b], sems[b])

    def process(j, b):
        base = wid * EPW + j * CH
        pltpu.make_async_copy(rcv.at[pl.ds(base, CH)], idx2.at[b],
                              sems[b]).wait()
        pltpu.sync_copy(tmp, accum.at[idx2.at[b]], add=True)

    issue(0, 0)

    def pair(k2, c):
        j0 = 2 * k2
        issue(j0 + 1, 1)
        process(j0, 0)
        issue(j0 + 2, 0)
        process(j0 + 1, 1)
        return c

    lax.fori_loop(0, (NCH - 1) // 2, pair, 0)
    process(NCH - 1, 0)
    plsc.subcore_barrier()
    for q in range(ROWS_T // CH):
        r0 = sid * ROWS_T + q * CH
        pltpu.sync_copy(accum.at[pl.ds(r0, CH)], tmp)
        pltpu.sync_copy(tmp, out.at[cid, pl.ds(r0, CH)])


def kernel(edge_idx, edge_features, node_features, params):
    snd = edge_idx[:, 0].astype(jnp.int32)
    rcv = edge_idx[:, 1].astype(jnp.int32)
    npart = len(PARTS)
    snd_h = [snd[_PART_OFF[i]:_PART_OFF[i + 1]] for i in range(npart)]
    rcv_h = [rcv[_PART_OFF[i]:_PART_OFF[i + 1]] for i in range(npart)]

    ef_h = [_mlp2(edge_features[_PART_OFF[i]:_PART_OFF[i + 1]],
                  params['embed_edge'], E_BLK) for i in range(npart)]
    nf = _mlp2(node_features, params['embed_node'], N_BLK)

    counts = _sc_counts(rcv)
    c0 = counts[0, :NNODES]
    c1 = counts[1, :NNODES]

    for step in params['process']:
        w1 = step['edge']['Ws'][0]
        nfa, nfb = _node_proj(nf, w1[:LATENT], w1[LATENT:2 * LATENT],
                              step['edge']['bs'][0])
        g_h = [_sc_gathers[i](nfa, nfb, snd_h[i], rcv_h[i])
               for i in range(npart)]
        new_ef_h = [_edge_step(g_h[i], ef_h[i], w1[2 * LATENT:],
                               step['edge'])
                    for i in range(npart)]
        parts_h = [_sc_scatters[i](rcv_h[i], new_ef_h[i])
                   for i in range(npart)]
        nf = _node_mlp(nf, parts_h, c0, c1, step['node'])
        ef_h = new_ef_h

    ef = jnp.concatenate([_mlp2(e, params['edge_out'], E_BLK) for e in ef_h],
                         axis=0)
    nf = _mlp2(nf, params['node_out'], N_BLK)
    return ef, nf
